# baseline (speedup 1.0000x reference)
import functools

import jax
import jax.numpy as jnp
from jax import lax
from jax.experimental import pallas as pl
from jax.experimental.pallas import tpu as pltpu
from jax.experimental.pallas import tpu_sc as plsc

NN = 10000
EE = 320000
FF = 128
GG = 64
NP_ = 10240
KC = 64
NTILES = 32
CPT = 160
EPAD = NTILES * CPT * KC
RB = 1000
NBLK = NN // RB


def _mesh():
    return plsc.VectorSubcoreMesh(core_axis_name="c", subcore_axis_name="s")


def _deg_body(dst_hbm, out_hbm, didx, ones_v, stage_v, acc, sa, sb):
    c = lax.axis_index("c")
    s = lax.axis_index("s")
    wid = s * 2 + c

    def fill_z(i, _):
        stage_v[pl.ds(i * 16, 16)] = jnp.zeros((16,), jnp.float32)
        return 0
    lax.fori_loop(0, 40, fill_z, 0)

    def fill_o(i, _):
        ones_v[pl.ds(i * 16, 16)] = jnp.full((16,), 1.0, jnp.float32)
        return 0
    lax.fori_loop(0, KC // 16, fill_o, 0)

    pltpu.sync_copy(stage_v, acc.at[pl.ds(s * 640, 640)])
    pltpu.sync_copy(dst_hbm.at[pl.ds(wid * CPT, CPT)], didx)
    plsc.subcore_barrier()

    def step(p, _):
        da = pltpu.async_copy(ones_v, acc.at[didx.at[2 * p]], sa, add=True)
        db = pltpu.async_copy(ones_v, acc.at[didx.at[2 * p + 1]], sb, add=True)
        da.wait()
        db.wait()
        return 0
    lax.fori_loop(0, CPT // 2, step, 0)

    plsc.subcore_barrier()
    pltpu.sync_copy(acc.at[pl.ds(s * 640, 640)], stage_v)
    pltpu.sync_copy(stage_v, out_hbm.at[pl.ds(c * NP_ + s * 640, 640)])


@jax.jit
def _sc_deg(dst2d):
    k = pl.kernel(
        _deg_body,
        out_type=jax.ShapeDtypeStruct((2 * NP_,), jnp.float32),
        mesh=_mesh(),
        scratch_types=[
            pltpu.VMEM((CPT, KC), jnp.int32),
            pltpu.VMEM((KC,), jnp.float32),
            pltpu.VMEM((640,), jnp.float32),
            pltpu.VMEM_SHARED((NP_,), jnp.float32),
            pltpu.SemaphoreType.DMA,
            pltpu.SemaphoreType.DMA,
        ],
    )
    return k(dst2d)


def _spmm_body(g_hbm, src_hbm, dst_hbm, out_hbm, sidx, didx, rowsa, rowsb,
               acc, sga, sgb, ssa, ssb):
    c = lax.axis_index("c")
    s = lax.axis_index("s")
    wid = s * 2 + c
    half = CPT // 2

    def fill_z(i, _):
        r = i // 8
        l = i - r * 8
        rowsa[r, pl.ds(l * 16, 16)] = jnp.zeros((16,), jnp.float32)
        return 0
    lax.fori_loop(0, 512, fill_z, 0)

    def zstripe(t, _):
        pltpu.sync_copy(rowsa, acc.at[pl.ds(s * 640 + t * 64, 64)])
        return 0
    lax.fori_loop(0, 10, zstripe, 0)
    plsc.subcore_barrier()

    def run_half(h, _):
        pltpu.sync_copy(src_hbm.at[pl.ds(wid * CPT + h * half, half)], sidx)
        pltpu.sync_copy(dst_hbm.at[pl.ds(wid * CPT + h * half, half)], didx)
        pltpu.async_copy(g_hbm.at[sidx.at[0]], rowsa, sga)

        def step(p, _):
            t0 = 2 * p
            pltpu.make_async_copy(g_hbm.at[sidx.at[t0]], rowsa, sga).wait()
            sa = pltpu.async_copy(rowsa, acc.at[didx.at[t0]], ssa, add=True)
            gb = pltpu.async_copy(g_hbm.at[sidx.at[t0 + 1]], rowsb, sgb)
            gb.wait()
            sb = pltpu.async_copy(rowsb, acc.at[didx.at[t0 + 1]], ssb, add=True)
            sa.wait()

            @pl.when(p < half // 2 - 1)
            def _():
                pltpu.async_copy(g_hbm.at[sidx.at[t0 + 2]], rowsa, sga)

            sb.wait()
            return 0
        lax.fori_loop(0, half // 2, step, 0)
        return 0
    lax.fori_loop(0, 2, run_half, 0)

    plsc.subcore_barrier()

    def wout(q, _):
        r0 = s * 640 + q * 128
        o0 = c * NP_ + r0
        ia = pltpu.async_copy(acc.at[pl.ds(r0, 64)], rowsa, sga)
        ib = pltpu.async_copy(acc.at[pl.ds(r0 + 64, 64)], rowsb, sgb)
        ia.wait()
        oa = pltpu.async_copy(rowsa, out_hbm.at[pl.ds(o0, 64)], ssa)
        ib.wait()
        ob = pltpu.async_copy(rowsb, out_hbm.at[pl.ds(o0 + 64, 64)], ssb)
        oa.wait()
        ob.wait()
        return 0
    lax.fori_loop(0, 5, wout, 0)


@jax.jit
def _sc_spmm(g, src2d, dst2d):
    k = pl.kernel(
        _spmm_body,
        out_type=jax.ShapeDtypeStruct((2 * NP_, FF), jnp.float32),
        mesh=_mesh(),
        scratch_types=[
            pltpu.VMEM((CPT // 2, KC), jnp.int32),
            pltpu.VMEM((CPT // 2, KC), jnp.int32),
            pltpu.VMEM((KC, FF), jnp.float32),
            pltpu.VMEM((KC, FF), jnp.float32),
            pltpu.VMEM_SHARED((NP_, FF), jnp.float32),
            pltpu.SemaphoreType.DMA,
            pltpu.SemaphoreType.DMA,
            pltpu.SemaphoreType.DMA,
            pltpu.SemaphoreType.DMA,
        ],
    )
    return k(g, src2d, dst2d)


def _tca_body(x_ref, w1_ref, d0_ref, d1_ref, rid_ref,
              h1_ref, g1_ref, dinv_ref, r1_ref):
    i = pl.program_id(0)
    xb = x_ref[...]
    h1 = jnp.dot(xb, w1_ref[...], preferred_element_type=jnp.float32)
    deg = d0_ref[...] + d1_ref[...] + 1.0
    dinv = lax.rsqrt(deg)
    h1_ref[...] = h1
    dinv_ref[...] = dinv
    g1_ref[...] = h1 * dinv
    rid = rid_ref[...]
    col = lax.broadcasted_iota(jnp.int32, (GG, RB), 1) + i * RB
    pmat = (rid == col).astype(jnp.float32)

    @pl.when(i == 0)
    def _():
        r1_ref[...] = jnp.zeros((GG, FF), jnp.float32)

    r1_ref[...] += jnp.dot(pmat, xb, preferred_element_type=jnp.float32)


@jax.jit
def _tc_a(x, W1, deg0, deg1, rid):
    return pl.pallas_call(
        _tca_body,
        grid=(NBLK,),
        in_specs=[
            pl.BlockSpec((RB, FF), lambda i: (i, 0)),
            pl.BlockSpec((FF, FF), lambda i: (0, 0)),
            pl.BlockSpec((RB, 1), lambda i: (i, 0)),
            pl.BlockSpec((RB, 1), lambda i: (i, 0)),
            pl.BlockSpec((GG, 1), lambda i: (0, 0)),
        ],
        out_specs=[
            pl.BlockSpec((RB, FF), lambda i: (i, 0)),
            pl.BlockSpec((RB, FF), lambda i: (i, 0)),
            pl.BlockSpec((RB, 1), lambda i: (i, 0)),
            pl.BlockSpec((GG, FF), lambda i: (0, 0)),
        ],
        out_shape=[
            jax.ShapeDtypeStruct((NN, FF), jnp.float32),
            jax.ShapeDtypeStruct((NN, FF), jnp.float32),
            jax.ShapeDtypeStruct((NN, 1), jnp.float32),
            jax.ShapeDtypeStruct((GG, FF), jnp.float32),
        ],
    )(x, W1, deg0, deg1, rid)


def _tcb_body(a0_ref, a1_ref, h1_ref, dinv_ref, b1_ref, bat_ref, rid_ref,
              r1_ref, w2a_ref, w2b_ref,
              g2_ref, h2l_ref, r2_ref, r1w_ref):
    i = pl.program_id(0)

    @pl.when(i == 0)
    def _():
        r1w_ref[...] = jnp.dot(jnp.maximum(r1_ref[...], 0.0), w2b_ref[...],
                               preferred_element_type=jnp.float32)
        r2_ref[...] = jnp.zeros((GG, FF), jnp.float32)

    dinv = dinv_ref[...]
    c1 = dinv * (a0_ref[...] + a1_ref[...]) + dinv * dinv * h1_ref[...] + b1_ref[...]
    relu1 = jnp.maximum(c1, 0.0)
    bat = bat_ref[...]
    bmat = (bat == lax.broadcasted_iota(jnp.int32, (RB, GG), 1)).astype(jnp.float32)
    h2 = (jnp.dot(relu1, w2a_ref[...], preferred_element_type=jnp.float32)
          + jnp.dot(bmat, r1w_ref[...], preferred_element_type=jnp.float32))
    h2l_ref[...] = h2
    g2_ref[...] = h2 * dinv

    rid = rid_ref[...]
    col = lax.broadcasted_iota(jnp.int32, (GG, RB), 1) + i * RB
    pmat = (rid == col).astype(jnp.float32)
    r2_ref[...] += jnp.dot(pmat, c1, preferred_element_type=jnp.float32)


@jax.jit
def _tc_b(a0, a1, h1, dinv, b1, bat, rid, roots1, w2a, w2b):
    return pl.pallas_call(
        _tcb_body,
        grid=(NBLK,),
        in_specs=[
            pl.BlockSpec((RB, FF), lambda i: (i, 0)),
            pl.BlockSpec((RB, FF), lambda i: (i, 0)),
            pl.BlockSpec((RB, FF), lambda i: (i, 0)),
            pl.BlockSpec((RB, 1), lambda i: (i, 0)),
            pl.BlockSpec((1, FF), lambda i: (0, 0)),
            pl.BlockSpec((RB, 1), lambda i: (i, 0)),
            pl.BlockSpec((GG, 1), lambda i: (0, 0)),
            pl.BlockSpec((GG, FF), lambda i: (0, 0)),
            pl.BlockSpec((FF, FF), lambda i: (0, 0)),
            pl.BlockSpec((FF, FF), lambda i: (0, 0)),
        ],
        out_specs=[
            pl.BlockSpec((RB, FF), lambda i: (i, 0)),
            pl.BlockSpec((RB, FF), lambda i: (i, 0)),
            pl.BlockSpec((GG, FF), lambda i: (0, 0)),
        ],
        out_shape=[
            jax.ShapeDtypeStruct((NN, FF), jnp.float32),
            jax.ShapeDtypeStruct((NN, FF), jnp.float32),
            jax.ShapeDtypeStruct((GG, FF), jnp.float32),
        ],
        scratch_shapes=[pltpu.VMEM((GG, FF), jnp.float32)],
    )(a0, a1, h1, dinv, b1, bat, rid, roots1, w2a, w2b)


def _tcc_body(a0_ref, a1_ref, h2l_ref, dinv_ref, b2_ref, bat_ref, r2_ref,
              out_ref, sum_ref, cnt_ref):
    i = pl.program_id(0)

    @pl.when(i == 0)
    def _():
        sum_ref[...] = jnp.zeros((GG, FF), jnp.float32)
        cnt_ref[...] = jnp.zeros((GG, 1), jnp.float32)

    dinv = dinv_ref[...]
    c2 = dinv * (a0_ref[...] + a1_ref[...]) + dinv * dinv * h2l_ref[...] + b2_ref[...]
    relu2 = jnp.maximum(c2, 0.0)
    bat = bat_ref[...]
    bmat = (bat == lax.broadcasted_iota(jnp.int32, (RB, GG), 1)).astype(jnp.float32)
    sum_ref[...] += lax.dot_general(bmat, relu2, (((0,), (0,)), ((), ())),
                                    preferred_element_type=jnp.float32)
    ones_col = jnp.ones((RB, 1), jnp.float32)
    cnt_ref[...] += lax.dot_general(bmat, ones_col, (((0,), (0,)), ((), ())),
                                    preferred_element_type=jnp.float32)

    @pl.when(i == NBLK - 1)
    def _():
        cnt = cnt_ref[...]
        mean = sum_ref[...] / jnp.maximum(cnt, 1.0)
        rootp = jnp.where(cnt > 0.0, r2_ref[...], 0.0)
        out_ref[...] = jnp.concatenate([mean, rootp], axis=1)


@jax.jit
def _tc_c(a0, a1, h2l, dinv, b2, bat, roots2):
    return pl.pallas_call(
        _tcc_body,
        grid=(NBLK,),
        in_specs=[
            pl.BlockSpec((RB, FF), lambda i: (i, 0)),
            pl.BlockSpec((RB, FF), lambda i: (i, 0)),
            pl.BlockSpec((RB, FF), lambda i: (i, 0)),
            pl.BlockSpec((RB, 1), lambda i: (i, 0)),
            pl.BlockSpec((1, FF), lambda i: (0, 0)),
            pl.BlockSpec((RB, 1), lambda i: (i, 0)),
            pl.BlockSpec((GG, FF), lambda i: (0, 0)),
        ],
        out_specs=pl.BlockSpec((GG, 2 * FF), lambda i: (0, 0)),
        out_shape=jax.ShapeDtypeStruct((GG, 2 * FF), jnp.float32),
        scratch_shapes=[
            pltpu.VMEM((GG, FF), jnp.float32),
            pltpu.VMEM((GG, 1), jnp.float32),
        ],
    )(a0, a1, h2l, dinv, b2, bat, roots2)


def kernel(x, edge_index, root_index, batch, W1, b1, W2, b2):
    x = x.astype(jnp.float32)
    src = edge_index[0]
    dst = edge_index[1]
    npad = EPAD - EE
    src_p = jnp.concatenate(
        [src, jnp.zeros((npad,), jnp.int32)]).reshape(NTILES * CPT, KC)
    dst_p = jnp.concatenate(
        [dst, NN + (jnp.arange(npad, dtype=jnp.int32) % (NP_ - NN))]
    ).reshape(NTILES * CPT, KC)

    degf = _sc_deg(dst_p)
    deg0 = degf[:NP_].reshape(NP_, 1)
    deg1 = degf[NP_:].reshape(NP_, 1)
    rid = root_index.reshape(GG, 1)
    bat = batch.reshape(NN, 1)

    h1, g1, dinv, roots1 = _tc_a(x, W1, deg0, deg1, rid)

    p1 = _sc_spmm(g1, src_p, dst_p)
    g2, h2l, roots2 = _tc_b(p1[:NP_], p1[NP_:], h1, dinv,
                            b1.reshape(1, FF), bat, rid, roots1,
                            W2[:FF], W2[FF:])

    p2 = _sc_spmm(g2, src_p, dst_p)
    out = _tc_c(p2[:NP_], p2[NP_:], h2l, dinv, b2.reshape(1, FF), bat, roots2)
    return out


# speedup vs baseline: 2.5109x; 2.5109x over previous
import functools

import jax
import jax.numpy as jnp
from jax import lax
from jax.experimental import pallas as pl
from jax.experimental.pallas import tpu as pltpu
from jax.experimental.pallas import tpu_sc as plsc

NN = 10000
EE = 320000
FF = 128
GG = 64
NP_ = 10240
KC = 64
NTILES = 32
CPT = 160
EPAD = NTILES * CPT * KC
RB = 1000
NBLK = NN // RB


def _mesh():
    return plsc.VectorSubcoreMesh(core_axis_name="c", subcore_axis_name="s")


def _deg_body(dst_hbm, out_hbm, didx, ones_v, stage_v, acc, sa, sb):
    c = lax.axis_index("c")
    s = lax.axis_index("s")
    wid = s * 2 + c

    def fill_z(i, _):
        stage_v[pl.ds(i * 16, 16)] = jnp.zeros((16,), jnp.float32)
        return 0
    lax.fori_loop(0, 40, fill_z, 0)

    def fill_o(i, _):
        ones_v[pl.ds(i * 16, 16)] = jnp.full((16,), 1.0, jnp.float32)
        return 0
    lax.fori_loop(0, KC // 16, fill_o, 0)

    pltpu.sync_copy(stage_v, acc.at[pl.ds(s * 640, 640)])
    pltpu.sync_copy(dst_hbm.at[pl.ds(wid * CPT, CPT)], didx)
    plsc.subcore_barrier()

    def step(p, _):
        da = pltpu.async_copy(ones_v, acc.at[didx.at[2 * p]], sa, add=True)
        db = pltpu.async_copy(ones_v, acc.at[didx.at[2 * p + 1]], sb, add=True)
        da.wait()
        db.wait()
        return 0
    lax.fori_loop(0, CPT // 2, step, 0)

    plsc.subcore_barrier()
    pltpu.sync_copy(acc.at[pl.ds(s * 640, 640)], stage_v)
    pltpu.sync_copy(stage_v, out_hbm.at[pl.ds(c * NP_ + s * 640, 640)])


@jax.jit
def _sc_deg(dst2d):
    k = pl.kernel(
        _deg_body,
        out_type=jax.ShapeDtypeStruct((2 * NP_,), jnp.float32),
        mesh=_mesh(),
        scratch_types=[
            pltpu.VMEM((CPT, KC), jnp.int32),
            pltpu.VMEM((KC,), jnp.float32),
            pltpu.VMEM((640,), jnp.float32),
            pltpu.VMEM_SHARED((NP_,), jnp.float32),
            pltpu.SemaphoreType.DMA,
            pltpu.SemaphoreType.DMA,
        ],
    )
    return k(dst2d)


def _spmm_body(g_hbm, src_hbm, dst_hbm, out_hbm, sidx, didx, rowsa, rowsb,
               acc, sga, sgb, ssa, ssb):
    c = lax.axis_index("c")
    s = lax.axis_index("s")
    wid = s * 2 + c
    half = CPT // 2

    def fill_z(i, _):
        r = i // 8
        l = i - r * 8
        rowsa[r, pl.ds(l * 16, 16)] = jnp.zeros((16,), jnp.float32)
        return 0
    lax.fori_loop(0, 512, fill_z, 0)

    def zstripe(t, _):
        pltpu.sync_copy(rowsa, acc.at[pl.ds(s * 640 + t * 64, 64)])
        return 0
    lax.fori_loop(0, 10, zstripe, 0)
    plsc.subcore_barrier()

    def run_half(h, _):
        pltpu.sync_copy(src_hbm.at[pl.ds(wid * CPT + h * half, half)], sidx)
        pltpu.sync_copy(dst_hbm.at[pl.ds(wid * CPT + h * half, half)], didx)
        pltpu.async_copy(g_hbm.at[sidx.at[0]], rowsa, sga)

        def step(p, _):
            t0 = 2 * p
            pltpu.make_async_copy(g_hbm.at[sidx.at[t0]], rowsa, sga).wait()
            sa = pltpu.async_copy(rowsa, acc.at[didx.at[t0]], ssa, add=True)
            gb = pltpu.async_copy(g_hbm.at[sidx.at[t0 + 1]], rowsb, sgb)
            gb.wait()
            sb = pltpu.async_copy(rowsb, acc.at[didx.at[t0 + 1]], ssb, add=True)
            sa.wait()

            @pl.when(p < half // 2 - 1)
            def _():
                pltpu.async_copy(g_hbm.at[sidx.at[t0 + 2]], rowsa, sga)

            sb.wait()
            return 0
        lax.fori_loop(0, half // 2, step, 0)
        return 0
    lax.fori_loop(0, 2, run_half, 0)

    plsc.subcore_barrier()

    def wout(q, _):
        r0 = s * 640 + q * 128
        o0 = c * NP_ + r0
        ia = pltpu.async_copy(acc.at[pl.ds(r0, 64)], rowsa, sga)
        ib = pltpu.async_copy(acc.at[pl.ds(r0 + 64, 64)], rowsb, sgb)
        ia.wait()
        oa = pltpu.async_copy(rowsa, out_hbm.at[pl.ds(o0, 64)], ssa)
        ib.wait()
        ob = pltpu.async_copy(rowsb, out_hbm.at[pl.ds(o0 + 64, 64)], ssb)
        oa.wait()
        ob.wait()
        return 0
    lax.fori_loop(0, 5, wout, 0)


@jax.jit
def _sc_spmm(g, src2d, dst2d):
    k = pl.kernel(
        _spmm_body,
        out_type=jax.ShapeDtypeStruct((2 * NP_, FF), jnp.float32),
        mesh=_mesh(),
        scratch_types=[
            pltpu.VMEM((CPT // 2, KC), jnp.int32),
            pltpu.VMEM((CPT // 2, KC), jnp.int32),
            pltpu.VMEM((KC, FF), jnp.float32),
            pltpu.VMEM((KC, FF), jnp.float32),
            pltpu.VMEM_SHARED((NP_, FF), jnp.float32),
            pltpu.SemaphoreType.DMA,
            pltpu.SemaphoreType.DMA,
            pltpu.SemaphoreType.DMA,
            pltpu.SemaphoreType.DMA,
        ],
    )
    return k(g, src2d, dst2d)


def _tca_body(x_ref, w1_ref, d0_ref, d1_ref, rid_ref,
              h1_ref, g1_ref, dinv_ref, r1_ref):
    i = pl.program_id(0)
    xb = x_ref[...]
    h1 = jnp.dot(xb, w1_ref[...], preferred_element_type=jnp.float32)
    deg = d0_ref[...] + d1_ref[...] + 1.0
    dinv = lax.rsqrt(deg)
    h1_ref[...] = h1
    dinv_ref[...] = dinv
    g1_ref[...] = h1 * dinv
    rid = rid_ref[...]
    col = lax.broadcasted_iota(jnp.int32, (GG, RB), 1) + i * RB
    pmat = (rid == col).astype(jnp.float32)

    @pl.when(i == 0)
    def _():
        r1_ref[...] = jnp.zeros((GG, FF), jnp.float32)

    r1_ref[...] += jnp.dot(pmat, xb, preferred_element_type=jnp.float32)


@jax.jit
def _tc_a(x, W1, deg0, deg1, rid):
    return pl.pallas_call(
        _tca_body,
        grid=(NBLK,),
        in_specs=[
            pl.BlockSpec((RB, FF), lambda i: (i, 0)),
            pl.BlockSpec((FF, FF), lambda i: (0, 0)),
            pl.BlockSpec((RB, 1), lambda i: (i, 0)),
            pl.BlockSpec((RB, 1), lambda i: (i, 0)),
            pl.BlockSpec((GG, 1), lambda i: (0, 0)),
        ],
        out_specs=[
            pl.BlockSpec((RB, FF), lambda i: (i, 0)),
            pl.BlockSpec((RB, FF), lambda i: (i, 0)),
            pl.BlockSpec((RB, 1), lambda i: (i, 0)),
            pl.BlockSpec((GG, FF), lambda i: (0, 0)),
        ],
        out_shape=[
            jax.ShapeDtypeStruct((NN, FF), jnp.float32),
            jax.ShapeDtypeStruct((NN, FF), jnp.float32),
            jax.ShapeDtypeStruct((NN, 1), jnp.float32),
            jax.ShapeDtypeStruct((GG, FF), jnp.float32),
        ],
    )(x, W1, deg0, deg1, rid)


def _tcb_body(a0_ref, a1_ref, h1_ref, dinv_ref, b1_ref, bat_ref, rid_ref,
              r1_ref, w2a_ref, w2b_ref,
              g2_ref, h2l_ref, r2_ref, r1w_ref):
    i = pl.program_id(0)

    @pl.when(i == 0)
    def _():
        r1w_ref[...] = jnp.dot(jnp.maximum(r1_ref[...], 0.0), w2b_ref[...],
                               preferred_element_type=jnp.float32)
        r2_ref[...] = jnp.zeros((GG, FF), jnp.float32)

    dinv = dinv_ref[...]
    c1 = dinv * (a0_ref[...] + a1_ref[...]) + dinv * dinv * h1_ref[...] + b1_ref[...]
    relu1 = jnp.maximum(c1, 0.0)
    bat = bat_ref[...]
    bmat = (bat == lax.broadcasted_iota(jnp.int32, (RB, GG), 1)).astype(jnp.float32)
    h2 = (jnp.dot(relu1, w2a_ref[...], preferred_element_type=jnp.float32)
          + jnp.dot(bmat, r1w_ref[...], preferred_element_type=jnp.float32))
    h2l_ref[...] = h2
    g2_ref[...] = h2 * dinv

    rid = rid_ref[...]
    col = lax.broadcasted_iota(jnp.int32, (GG, RB), 1) + i * RB
    pmat = (rid == col).astype(jnp.float32)
    r2_ref[...] += jnp.dot(pmat, c1, preferred_element_type=jnp.float32)


@jax.jit
def _tc_b(a0, a1, h1, dinv, b1, bat, rid, roots1, w2a, w2b):
    return pl.pallas_call(
        _tcb_body,
        grid=(NBLK,),
        in_specs=[
            pl.BlockSpec((RB, FF), lambda i: (i, 0)),
            pl.BlockSpec((RB, FF), lambda i: (i, 0)),
            pl.BlockSpec((RB, FF), lambda i: (i, 0)),
            pl.BlockSpec((RB, 1), lambda i: (i, 0)),
            pl.BlockSpec((1, FF), lambda i: (0, 0)),
            pl.BlockSpec((RB, 1), lambda i: (i, 0)),
            pl.BlockSpec((GG, 1), lambda i: (0, 0)),
            pl.BlockSpec((GG, FF), lambda i: (0, 0)),
            pl.BlockSpec((FF, FF), lambda i: (0, 0)),
            pl.BlockSpec((FF, FF), lambda i: (0, 0)),
        ],
        out_specs=[
            pl.BlockSpec((RB, FF), lambda i: (i, 0)),
            pl.BlockSpec((RB, FF), lambda i: (i, 0)),
            pl.BlockSpec((GG, FF), lambda i: (0, 0)),
        ],
        out_shape=[
            jax.ShapeDtypeStruct((NN, FF), jnp.float32),
            jax.ShapeDtypeStruct((NN, FF), jnp.float32),
            jax.ShapeDtypeStruct((GG, FF), jnp.float32),
        ],
        scratch_shapes=[pltpu.VMEM((GG, FF), jnp.float32)],
    )(a0, a1, h1, dinv, b1, bat, rid, roots1, w2a, w2b)


def _tcc_body(a0_ref, a1_ref, h2l_ref, dinv_ref, b2_ref, bat_ref, r2_ref,
              out_ref, sum_ref, cnt_ref):
    i = pl.program_id(0)

    @pl.when(i == 0)
    def _():
        sum_ref[...] = jnp.zeros((GG, FF), jnp.float32)
        cnt_ref[...] = jnp.zeros((GG, 1), jnp.float32)

    dinv = dinv_ref[...]
    c2 = dinv * (a0_ref[...] + a1_ref[...]) + dinv * dinv * h2l_ref[...] + b2_ref[...]
    relu2 = jnp.maximum(c2, 0.0)
    bat = bat_ref[...]
    bmat = (bat == lax.broadcasted_iota(jnp.int32, (RB, GG), 1)).astype(jnp.float32)
    sum_ref[...] += lax.dot_general(bmat, relu2, (((0,), (0,)), ((), ())),
                                    preferred_element_type=jnp.float32)
    ones_col = jnp.ones((RB, 1), jnp.float32)
    cnt_ref[...] += lax.dot_general(bmat, ones_col, (((0,), (0,)), ((), ())),
                                    preferred_element_type=jnp.float32)

    @pl.when(i == NBLK - 1)
    def _():
        cnt = cnt_ref[...]
        mean = sum_ref[...] / jnp.maximum(cnt, 1.0)
        rootp = jnp.where(cnt > 0.0, r2_ref[...], 0.0)
        out_ref[...] = jnp.concatenate([mean, rootp], axis=1)


@jax.jit
def _tc_c(a0, a1, h2l, dinv, b2, bat, roots2):
    return pl.pallas_call(
        _tcc_body,
        grid=(NBLK,),
        in_specs=[
            pl.BlockSpec((RB, FF), lambda i: (i, 0)),
            pl.BlockSpec((RB, FF), lambda i: (i, 0)),
            pl.BlockSpec((RB, FF), lambda i: (i, 0)),
            pl.BlockSpec((RB, 1), lambda i: (i, 0)),
            pl.BlockSpec((1, FF), lambda i: (0, 0)),
            pl.BlockSpec((RB, 1), lambda i: (i, 0)),
            pl.BlockSpec((GG, FF), lambda i: (0, 0)),
        ],
        out_specs=pl.BlockSpec((GG, 2 * FF), lambda i: (0, 0)),
        out_shape=jax.ShapeDtypeStruct((GG, 2 * FF), jnp.float32),
        scratch_shapes=[
            pltpu.VMEM((GG, FF), jnp.float32),
            pltpu.VMEM((GG, 1), jnp.float32),
        ],
    )(a0, a1, h2l, dinv, b2, bat, roots2)


def kernel(x, edge_index, root_index, batch, W1, b1, W2, b2):
    x = x.astype(jnp.float32)
    src = edge_index[0]
    dst = edge_index[1]
    npad = EPAD - EE
    src_p = jnp.concatenate(
        [src, jnp.arange(npad, dtype=jnp.int32) % NN]).reshape(NTILES * CPT, KC)
    dst_p = jnp.concatenate(
        [dst, NN + (jnp.arange(npad, dtype=jnp.int32) % (NP_ - NN))]
    ).reshape(NTILES * CPT, KC)

    degf = _sc_deg(dst_p)
    deg0 = degf[:NP_].reshape(NP_, 1)
    deg1 = degf[NP_:].reshape(NP_, 1)
    rid = root_index.reshape(GG, 1)
    bat = batch.reshape(NN, 1)

    h1, g1, dinv, roots1 = _tc_a(x, W1, deg0, deg1, rid)

    p1 = _sc_spmm(g1, src_p, dst_p)
    g2, h2l, roots2 = _tc_b(p1[:NP_], p1[NP_:], h1, dinv,
                            b1.reshape(1, FF), bat, rid, roots1,
                            W2[:FF], W2[FF:])

    p2 = _sc_spmm(g2, src_p, dst_p)
    out = _tc_c(p2[:NP_], p2[NP_:], h2l, dinv, b2.reshape(1, FF), bat, roots2)
    return out


# speedup vs baseline: 2.7485x; 1.0946x over previous
import functools

import jax
import jax.numpy as jnp
from jax import lax
from jax.experimental import pallas as pl
from jax.experimental.pallas import tpu as pltpu
from jax.experimental.pallas import tpu_sc as plsc

NN = 10000
EE = 320000
FF = 128
GG = 64
NP_ = 10240
KC = 80
NTILES = 32
CPT = 128
EPAD = NTILES * CPT * KC
RB = 1000
NBLK = NN // RB


def _mesh():
    return plsc.VectorSubcoreMesh(core_axis_name="c", subcore_axis_name="s")


def _deg_body(dst_hbm, out_hbm, didx, ones_v, stage_v, acc, sa, sb):
    c = lax.axis_index("c")
    s = lax.axis_index("s")
    wid = s * 2 + c

    def fill_z(i, _):
        stage_v[pl.ds(i * 16, 16)] = jnp.zeros((16,), jnp.float32)
        return 0
    lax.fori_loop(0, 40, fill_z, 0)

    def fill_o(i, _):
        ones_v[pl.ds(i * 16, 16)] = jnp.full((16,), 1.0, jnp.float32)
        return 0
    lax.fori_loop(0, KC // 16, fill_o, 0)

    pltpu.sync_copy(stage_v, acc.at[pl.ds(s * 640, 640)])
    pltpu.sync_copy(dst_hbm.at[pl.ds(wid * CPT, CPT)], didx)
    plsc.subcore_barrier()

    def step(p, _):
        da = pltpu.async_copy(ones_v, acc.at[didx.at[2 * p]], sa, add=True)
        db = pltpu.async_copy(ones_v, acc.at[didx.at[2 * p + 1]], sb, add=True)
        da.wait()
        db.wait()
        return 0
    lax.fori_loop(0, CPT // 2, step, 0)

    plsc.subcore_barrier()
    pltpu.sync_copy(acc.at[pl.ds(s * 640, 640)], stage_v)
    pltpu.sync_copy(stage_v, out_hbm.at[pl.ds(c * NP_ + s * 640, 640)])


@jax.jit
def _sc_deg(dst2d):
    k = pl.kernel(
        _deg_body,
        out_type=jax.ShapeDtypeStruct((2 * NP_,), jnp.float32),
        mesh=_mesh(),
        scratch_types=[
            pltpu.VMEM((CPT, KC), jnp.int32),
            pltpu.VMEM((KC,), jnp.float32),
            pltpu.VMEM((640,), jnp.float32),
            pltpu.VMEM_SHARED((NP_,), jnp.float32),
            pltpu.SemaphoreType.DMA,
            pltpu.SemaphoreType.DMA,
        ],
    )
    return k(dst2d)


def _spmm_body(g_hbm, src_hbm, dst_hbm, out_hbm, sidx, didx, rowsa, rowsb,
               acc, sga, sgb, ssa, ssb):
    c = lax.axis_index("c")
    s = lax.axis_index("s")
    wid = s * 2 + c
    half = CPT // 2

    def fill_z(i, _):
        r = i // 8
        l = i - r * 8
        rowsa[r, pl.ds(l * 16, 16)] = jnp.zeros((16,), jnp.float32)
        return 0
    lax.fori_loop(0, KC * 8, fill_z, 0)

    def zstripe(t, _):
        pltpu.sync_copy(rowsa, acc.at[pl.ds(s * 640 + t * KC, KC)])
        return 0
    lax.fori_loop(0, 640 // KC, zstripe, 0)
    plsc.subcore_barrier()

    def run_half(h, _):
        pltpu.sync_copy(src_hbm.at[pl.ds(wid * CPT + h * half, half)], sidx)
        pltpu.sync_copy(dst_hbm.at[pl.ds(wid * CPT + h * half, half)], didx)
        pltpu.async_copy(g_hbm.at[sidx.at[0]], rowsa, sga)

        def step(p, _):
            t0 = 2 * p
            pltpu.make_async_copy(g_hbm.at[sidx.at[t0]], rowsa, sga).wait()
            sa = pltpu.async_copy(rowsa, acc.at[didx.at[t0]], ssa, add=True)
            gb = pltpu.async_copy(g_hbm.at[sidx.at[t0 + 1]], rowsb, sgb)
            gb.wait()
            sb = pltpu.async_copy(rowsb, acc.at[didx.at[t0 + 1]], ssb, add=True)
            sa.wait()

            @pl.when(p < half // 2 - 1)
            def _():
                pltpu.async_copy(g_hbm.at[sidx.at[t0 + 2]], rowsa, sga)

            sb.wait()
            return 0
        lax.fori_loop(0, half // 2, step, 0)
        return 0
    lax.fori_loop(0, 2, run_half, 0)

    plsc.subcore_barrier()

    def wout(q, _):
        r0 = s * 640 + q * (2 * KC)
        o0 = c * NP_ + r0
        ia = pltpu.async_copy(acc.at[pl.ds(r0, KC)], rowsa, sga)
        ib = pltpu.async_copy(acc.at[pl.ds(r0 + KC, KC)], rowsb, sgb)
        ia.wait()
        oa = pltpu.async_copy(rowsa, out_hbm.at[pl.ds(o0, KC)], ssa)
        ib.wait()
        ob = pltpu.async_copy(rowsb, out_hbm.at[pl.ds(o0 + KC, KC)], ssb)
        oa.wait()
        ob.wait()
        return 0
    lax.fori_loop(0, 320 // KC, wout, 0)


@jax.jit
def _sc_spmm(g, src2d, dst2d):
    k = pl.kernel(
        _spmm_body,
        out_type=jax.ShapeDtypeStruct((2 * NP_, FF), jnp.float32),
        mesh=_mesh(),
        scratch_types=[
            pltpu.VMEM((CPT // 2, KC), jnp.int32),
            pltpu.VMEM((CPT // 2, KC), jnp.int32),
            pltpu.VMEM((KC, FF), jnp.float32),
            pltpu.VMEM((KC, FF), jnp.float32),
            pltpu.VMEM_SHARED((NP_, FF), jnp.float32),
            pltpu.SemaphoreType.DMA,
            pltpu.SemaphoreType.DMA,
            pltpu.SemaphoreType.DMA,
            pltpu.SemaphoreType.DMA,
        ],
    )
    return k(g, src2d, dst2d)


def _tca_body(x_ref, w1_ref, d0_ref, d1_ref, rid_ref,
              h1_ref, g1_ref, dinv_ref, r1_ref):
    i = pl.program_id(0)
    xb = x_ref[...]
    h1 = jnp.dot(xb, w1_ref[...], preferred_element_type=jnp.float32)
    deg = d0_ref[...] + d1_ref[...] + 1.0
    dinv = lax.rsqrt(deg)
    h1_ref[...] = h1
    dinv_ref[...] = dinv
    g1_ref[...] = h1 * dinv
    rid = rid_ref[...]
    col = lax.broadcasted_iota(jnp.int32, (GG, RB), 1) + i * RB
    pmat = (rid == col).astype(jnp.float32)

    @pl.when(i == 0)
    def _():
        r1_ref[...] = jnp.zeros((GG, FF), jnp.float32)

    r1_ref[...] += jnp.dot(pmat, xb, preferred_element_type=jnp.float32)


@jax.jit
def _tc_a(x, W1, deg0, deg1, rid):
    return pl.pallas_call(
        _tca_body,
        grid=(NBLK,),
        in_specs=[
            pl.BlockSpec((RB, FF), lambda i: (i, 0)),
            pl.BlockSpec((FF, FF), lambda i: (0, 0)),
            pl.BlockSpec((RB, 1), lambda i: (i, 0)),
            pl.BlockSpec((RB, 1), lambda i: (i, 0)),
            pl.BlockSpec((GG, 1), lambda i: (0, 0)),
        ],
        out_specs=[
            pl.BlockSpec((RB, FF), lambda i: (i, 0)),
            pl.BlockSpec((RB, FF), lambda i: (i, 0)),
            pl.BlockSpec((RB, 1), lambda i: (i, 0)),
            pl.BlockSpec((GG, FF), lambda i: (0, 0)),
        ],
        out_shape=[
            jax.ShapeDtypeStruct((NN, FF), jnp.float32),
            jax.ShapeDtypeStruct((NN, FF), jnp.float32),
            jax.ShapeDtypeStruct((NN, 1), jnp.float32),
            jax.ShapeDtypeStruct((GG, FF), jnp.float32),
        ],
    )(x, W1, deg0, deg1, rid)


def _tcb_body(a0_ref, a1_ref, h1_ref, dinv_ref, b1_ref, bat_ref, rid_ref,
              r1_ref, w2a_ref, w2b_ref,
              g2_ref, h2l_ref, r2_ref, r1w_ref):
    i = pl.program_id(0)

    @pl.when(i == 0)
    def _():
        r1w_ref[...] = jnp.dot(jnp.maximum(r1_ref[...], 0.0), w2b_ref[...],
                               preferred_element_type=jnp.float32)
        r2_ref[...] = jnp.zeros((GG, FF), jnp.float32)

    dinv = dinv_ref[...]
    c1 = dinv * (a0_ref[...] + a1_ref[...]) + dinv * dinv * h1_ref[...] + b1_ref[...]
    relu1 = jnp.maximum(c1, 0.0)
    bat = bat_ref[...]
    bmat = (bat == lax.broadcasted_iota(jnp.int32, (RB, GG), 1)).astype(jnp.float32)
    h2 = (jnp.dot(relu1, w2a_ref[...], preferred_element_type=jnp.float32)
          + jnp.dot(bmat, r1w_ref[...], preferred_element_type=jnp.float32))
    h2l_ref[...] = h2
    g2_ref[...] = h2 * dinv

    rid = rid_ref[...]
    col = lax.broadcasted_iota(jnp.int32, (GG, RB), 1) + i * RB
    pmat = (rid == col).astype(jnp.float32)
    r2_ref[...] += jnp.dot(pmat, c1, preferred_element_type=jnp.float32)


@jax.jit
def _tc_b(a0, a1, h1, dinv, b1, bat, rid, roots1, w2a, w2b):
    return pl.pallas_call(
        _tcb_body,
        grid=(NBLK,),
        in_specs=[
            pl.BlockSpec((RB, FF), lambda i: (i, 0)),
            pl.BlockSpec((RB, FF), lambda i: (i, 0)),
            pl.BlockSpec((RB, FF), lambda i: (i, 0)),
            pl.BlockSpec((RB, 1), lambda i: (i, 0)),
            pl.BlockSpec((1, FF), lambda i: (0, 0)),
            pl.BlockSpec((RB, 1), lambda i: (i, 0)),
            pl.BlockSpec((GG, 1), lambda i: (0, 0)),
            pl.BlockSpec((GG, FF), lambda i: (0, 0)),
            pl.BlockSpec((FF, FF), lambda i: (0, 0)),
            pl.BlockSpec((FF, FF), lambda i: (0, 0)),
        ],
        out_specs=[
            pl.BlockSpec((RB, FF), lambda i: (i, 0)),
            pl.BlockSpec((RB, FF), lambda i: (i, 0)),
            pl.BlockSpec((GG, FF), lambda i: (0, 0)),
        ],
        out_shape=[
            jax.ShapeDtypeStruct((NN, FF), jnp.float32),
            jax.ShapeDtypeStruct((NN, FF), jnp.float32),
            jax.ShapeDtypeStruct((GG, FF), jnp.float32),
        ],
        scratch_shapes=[pltpu.VMEM((GG, FF), jnp.float32)],
    )(a0, a1, h1, dinv, b1, bat, rid, roots1, w2a, w2b)


def _tcc_body(a0_ref, a1_ref, h2l_ref, dinv_ref, b2_ref, bat_ref, r2_ref,
              out_ref, sum_ref, cnt_ref):
    i = pl.program_id(0)

    @pl.when(i == 0)
    def _():
        sum_ref[...] = jnp.zeros((GG, FF), jnp.float32)
        cnt_ref[...] = jnp.zeros((GG, 1), jnp.float32)

    dinv = dinv_ref[...]
    c2 = dinv * (a0_ref[...] + a1_ref[...]) + dinv * dinv * h2l_ref[...] + b2_ref[...]
    relu2 = jnp.maximum(c2, 0.0)
    bat = bat_ref[...]
    bmat = (bat == lax.broadcasted_iota(jnp.int32, (RB, GG), 1)).astype(jnp.float32)
    sum_ref[...] += lax.dot_general(bmat, relu2, (((0,), (0,)), ((), ())),
                                    preferred_element_type=jnp.float32)
    ones_col = jnp.ones((RB, 1), jnp.float32)
    cnt_ref[...] += lax.dot_general(bmat, ones_col, (((0,), (0,)), ((), ())),
                                    preferred_element_type=jnp.float32)

    @pl.when(i == NBLK - 1)
    def _():
        cnt = cnt_ref[...]
        mean = sum_ref[...] / jnp.maximum(cnt, 1.0)
        rootp = jnp.where(cnt > 0.0, r2_ref[...], 0.0)
        out_ref[...] = jnp.concatenate([mean, rootp], axis=1)


@jax.jit
def _tc_c(a0, a1, h2l, dinv, b2, bat, roots2):
    return pl.pallas_call(
        _tcc_body,
        grid=(NBLK,),
        in_specs=[
            pl.BlockSpec((RB, FF), lambda i: (i, 0)),
            pl.BlockSpec((RB, FF), lambda i: (i, 0)),
            pl.BlockSpec((RB, FF), lambda i: (i, 0)),
            pl.BlockSpec((RB, 1), lambda i: (i, 0)),
            pl.BlockSpec((1, FF), lambda i: (0, 0)),
            pl.BlockSpec((RB, 1), lambda i: (i, 0)),
            pl.BlockSpec((GG, FF), lambda i: (0, 0)),
        ],
        out_specs=pl.BlockSpec((GG, 2 * FF), lambda i: (0, 0)),
        out_shape=jax.ShapeDtypeStruct((GG, 2 * FF), jnp.float32),
        scratch_shapes=[
            pltpu.VMEM((GG, FF), jnp.float32),
            pltpu.VMEM((GG, 1), jnp.float32),
        ],
    )(a0, a1, h2l, dinv, b2, bat, roots2)


def kernel(x, edge_index, root_index, batch, W1, b1, W2, b2):
    x = x.astype(jnp.float32)
    src = edge_index[0]
    dst = edge_index[1]
    npad = EPAD - EE
    src_p = jnp.concatenate(
        [src, jnp.arange(npad, dtype=jnp.int32) % NN]).reshape(NTILES * CPT, KC)
    dst_p = jnp.concatenate(
        [dst, NN + (jnp.arange(npad, dtype=jnp.int32) % (NP_ - NN))]
    ).reshape(NTILES * CPT, KC)

    degf = _sc_deg(dst_p)
    deg0 = degf[:NP_].reshape(NP_, 1)
    deg1 = degf[NP_:].reshape(NP_, 1)
    rid = root_index.reshape(GG, 1)
    bat = batch.reshape(NN, 1)

    h1, g1, dinv, roots1 = _tc_a(x, W1, deg0, deg1, rid)

    p1 = _sc_spmm(g1, src_p, dst_p)
    g2, h2l, roots2 = _tc_b(p1[:NP_], p1[NP_:], h1, dinv,
                            b1.reshape(1, FF), bat, rid, roots1,
                            W2[:FF], W2[FF:])

    p2 = _sc_spmm(g2, src_p, dst_p)
    out = _tc_c(p2[:NP_], p2[NP_:], h2l, dinv, b2.reshape(1, FF), bat, roots2)
    return out


# speedup vs baseline: 2.9624x; 1.0778x over previous
import functools

import jax
import jax.numpy as jnp
from jax import lax
from jax.experimental import pallas as pl
from jax.experimental.pallas import tpu as pltpu
from jax.experimental.pallas import tpu_sc as plsc

NN = 10000
EE = 320000
FF = 128
GG = 64
NP_ = 10240
KC = 80
NTILES = 32
CPT = 125
RB = 2000
NBLK = NN // RB


def _mesh():
    return plsc.VectorSubcoreMesh(core_axis_name="c", subcore_axis_name="s")


def _deg_body(dst_hbm, out_hbm, didx, ones_v, stage_v, acc, sa, sb):
    c = lax.axis_index("c")
    s = lax.axis_index("s")
    wid = s * 2 + c

    def fill_z(i, _):
        stage_v[pl.ds(i * 16, 16)] = jnp.zeros((16,), jnp.float32)
        return 0
    lax.fori_loop(0, 40, fill_z, 0)

    def fill_o(i, _):
        ones_v[pl.ds(i * 16, 16)] = jnp.full((16,), 1.0, jnp.float32)
        return 0
    lax.fori_loop(0, KC // 16, fill_o, 0)

    pltpu.sync_copy(stage_v, acc.at[pl.ds(s * 640, 640)])
    base = wid * CPT
    ab = pl.multiple_of(
        jnp.minimum((base // 8) * 8, NTILES * CPT - 136), 8)
    off = base - ab
    pltpu.sync_copy(dst_hbm.at[1, pl.ds(ab, 136)], didx)
    plsc.subcore_barrier()

    def step(p, _):
        da = pltpu.async_copy(ones_v, acc.at[didx.at[off + 2 * p]], sa, add=True)
        db = pltpu.async_copy(ones_v, acc.at[didx.at[off + 2 * p + 1]], sb, add=True)
        da.wait()
        db.wait()
        return 0
    lax.fori_loop(0, CPT // 2, step, 0)
    dl = pltpu.async_copy(ones_v, acc.at[didx.at[off + CPT - 1]], sa, add=True)
    dl.wait()

    plsc.subcore_barrier()
    pltpu.sync_copy(acc.at[pl.ds(s * 640, 640)], stage_v)
    pltpu.sync_copy(stage_v, out_hbm.at[pl.ds(c * NP_ + s * 640, 640)])


@jax.jit
def _sc_deg(dst2d):
    k = pl.kernel(
        _deg_body,
        out_type=jax.ShapeDtypeStruct((2 * NP_,), jnp.float32),
        mesh=_mesh(),
        scratch_types=[
            pltpu.VMEM((136, KC), jnp.int32),
            pltpu.VMEM((KC,), jnp.float32),
            pltpu.VMEM((640,), jnp.float32),
            pltpu.VMEM_SHARED((NP_,), jnp.float32),
            pltpu.SemaphoreType.DMA,
            pltpu.SemaphoreType.DMA,
        ],
    )
    return k(dst2d)


def _spmm_body(g_hbm, src_hbm, out_hbm, sidx, didx, rowsa, rowsb,
               acc, sga, sgb, ssa, ssb):
    dst_hbm = src_hbm
    c = lax.axis_index("c")
    s = lax.axis_index("s")
    wid = s * 2 + c

    def fill_z(i, _):
        r = i // 8
        l = i - r * 8
        rowsa[r, pl.ds(l * 16, 16)] = jnp.zeros((16,), jnp.float32)
        return 0
    lax.fori_loop(0, KC * 8, fill_z, 0)

    def zstripe(t, _):
        pltpu.sync_copy(rowsa, acc.at[pl.ds(s * 640 + t * KC, KC)])
        return 0
    lax.fori_loop(0, 640 // KC, zstripe, 0)
    plsc.subcore_barrier()

    for h in (0, 1):
        base = wid * CPT + h * 62
        ab = pl.multiple_of(
            jnp.minimum((base // 8) * 8, NTILES * CPT - 72), 8)
        off = base - ab
        pltpu.sync_copy(src_hbm.at[0, pl.ds(ab, 72)], sidx)
        pltpu.sync_copy(src_hbm.at[1, pl.ds(ab, 72)], didx)
        pltpu.async_copy(g_hbm.at[sidx.at[off]], rowsa, sga)

        def step(p, _):
            t0 = off + 2 * p
            pltpu.make_async_copy(g_hbm.at[sidx.at[t0]], rowsa, sga).wait()
            sa = pltpu.async_copy(rowsa, acc.at[didx.at[t0]], ssa, add=True)
            gb = pltpu.async_copy(g_hbm.at[sidx.at[t0 + 1]], rowsb, sgb)
            gb.wait()
            sb = pltpu.async_copy(rowsb, acc.at[didx.at[t0 + 1]], ssb, add=True)
            sa.wait()

            @pl.when(p < 30 + h)
            def _():
                pltpu.async_copy(g_hbm.at[sidx.at[t0 + 2]], rowsa, sga)

            sb.wait()
            return 0
        lax.fori_loop(0, 31, step, 0)
        if h == 1:
            pltpu.make_async_copy(g_hbm.at[sidx.at[off + 62]], rowsa, sga).wait()
            sl = pltpu.async_copy(rowsa, acc.at[didx.at[off + 62]], ssa, add=True)
            sl.wait()

    plsc.subcore_barrier()

    def wout(q, _):
        r0 = s * 640 + q * (2 * KC)
        o0 = c * NP_ + r0
        ia = pltpu.async_copy(acc.at[pl.ds(r0, KC)], rowsa, sga)
        ib = pltpu.async_copy(acc.at[pl.ds(r0 + KC, KC)], rowsb, sgb)
        ia.wait()
        oa = pltpu.async_copy(rowsa, out_hbm.at[pl.ds(o0, KC)], ssa)
        ib.wait()
        ob = pltpu.async_copy(rowsb, out_hbm.at[pl.ds(o0 + KC, KC)], ssb)
        oa.wait()
        ob.wait()
        return 0
    lax.fori_loop(0, 320 // KC, wout, 0)


@jax.jit
def _sc_spmm(g, ei3):
    k = pl.kernel(
        _spmm_body,
        out_type=jax.ShapeDtypeStruct((2 * NP_, FF), jnp.float32),
        mesh=_mesh(),
        scratch_types=[
            pltpu.VMEM((72, KC), jnp.int32),
            pltpu.VMEM((72, KC), jnp.int32),
            pltpu.VMEM((KC, FF), jnp.float32),
            pltpu.VMEM((KC, FF), jnp.float32),
            pltpu.VMEM_SHARED((NP_, FF), jnp.float32),
            pltpu.SemaphoreType.DMA,
            pltpu.SemaphoreType.DMA,
            pltpu.SemaphoreType.DMA,
            pltpu.SemaphoreType.DMA,
        ],
    )
    return k(g, ei3)


def _tca_body(x_ref, w1_ref, d0_ref, d1_ref, rid_ref,
              h1_ref, g1_ref, dinv_ref, r1_ref):
    i = pl.program_id(0)
    xb = x_ref[...]
    h1 = jnp.dot(xb, w1_ref[...], preferred_element_type=jnp.float32)
    deg = (d0_ref[...] + d1_ref[...]).reshape(RB, 1) + 1.0
    dinv = lax.rsqrt(deg)
    h1_ref[...] = h1
    dinv_ref[...] = dinv
    g1_ref[...] = h1 * dinv
    rid = rid_ref[...]
    col = lax.broadcasted_iota(jnp.int32, (GG, RB), 1) + i * RB
    pmat = (rid == col).astype(jnp.float32)

    @pl.when(i == 0)
    def _():
        r1_ref[...] = jnp.zeros((GG, FF), jnp.float32)

    r1_ref[...] += jnp.dot(pmat, xb, preferred_element_type=jnp.float32)


@jax.jit
def _tc_a(x, W1, degp, rid):
    return pl.pallas_call(
        _tca_body,
        grid=(NBLK,),
        in_specs=[
            pl.BlockSpec((RB, FF), lambda i: (i, 0)),
            pl.BlockSpec((FF, FF), lambda i: (0, 0)),
            pl.BlockSpec((1, RB, 1), lambda i: (0, i, 0)),
            pl.BlockSpec((1, RB, 1), lambda i: (1, i, 0)),
            pl.BlockSpec((GG, 1), lambda i: (0, 0)),
        ],
        out_specs=[
            pl.BlockSpec((RB, FF), lambda i: (i, 0)),
            pl.BlockSpec((RB, FF), lambda i: (i, 0)),
            pl.BlockSpec((RB, 1), lambda i: (i, 0)),
            pl.BlockSpec((GG, FF), lambda i: (0, 0)),
        ],
        out_shape=[
            jax.ShapeDtypeStruct((NN, FF), jnp.float32),
            jax.ShapeDtypeStruct((NN, FF), jnp.float32),
            jax.ShapeDtypeStruct((NN, 1), jnp.float32),
            jax.ShapeDtypeStruct((GG, FF), jnp.float32),
        ],
    )(x, W1, degp, degp, rid)


def _tcb_body(a0_ref, a1_ref, h1_ref, dinv_ref, b1_ref, bat_ref, rid_ref,
              r1_ref, w2a_ref, w2b_ref,
              g2_ref, h2l_ref, r2_ref, r1w_ref):
    i = pl.program_id(0)

    @pl.when(i == 0)
    def _():
        r1w_ref[...] = jnp.dot(jnp.maximum(r1_ref[...], 0.0), w2b_ref[...],
                               preferred_element_type=jnp.float32)
        r2_ref[...] = jnp.zeros((GG, FF), jnp.float32)

    dinv = dinv_ref[...]
    accs = (a0_ref[...] + a1_ref[...]).reshape(RB, FF)
    c1 = dinv * accs + dinv * dinv * h1_ref[...] + b1_ref[...]
    relu1 = jnp.maximum(c1, 0.0)
    bat = bat_ref[...]
    bmat = (bat == lax.broadcasted_iota(jnp.int32, (RB, GG), 1)).astype(jnp.float32)
    h2 = (jnp.dot(relu1, w2a_ref[...], preferred_element_type=jnp.float32)
          + jnp.dot(bmat, r1w_ref[...], preferred_element_type=jnp.float32))
    h2l_ref[...] = h2
    g2_ref[...] = h2 * dinv

    rid = rid_ref[...]
    col = lax.broadcasted_iota(jnp.int32, (GG, RB), 1) + i * RB
    pmat = (rid == col).astype(jnp.float32)
    r2_ref[...] += jnp.dot(pmat, c1, preferred_element_type=jnp.float32)


@jax.jit
def _tc_b(p1, h1, dinv, b1, bat, rid, roots1, w2a, w2b):
    return pl.pallas_call(
        _tcb_body,
        grid=(NBLK,),
        in_specs=[
            pl.BlockSpec((1, RB, FF), lambda i: (0, i, 0)),
            pl.BlockSpec((1, RB, FF), lambda i: (1, i, 0)),
            pl.BlockSpec((RB, FF), lambda i: (i, 0)),
            pl.BlockSpec((RB, 1), lambda i: (i, 0)),
            pl.BlockSpec((1, FF), lambda i: (0, 0)),
            pl.BlockSpec((RB, 1), lambda i: (i, 0)),
            pl.BlockSpec((GG, 1), lambda i: (0, 0)),
            pl.BlockSpec((GG, FF), lambda i: (0, 0)),
            pl.BlockSpec((FF, FF), lambda i: (0, 0)),
            pl.BlockSpec((FF, FF), lambda i: (0, 0)),
        ],
        out_specs=[
            pl.BlockSpec((RB, FF), lambda i: (i, 0)),
            pl.BlockSpec((RB, FF), lambda i: (i, 0)),
            pl.BlockSpec((GG, FF), lambda i: (0, 0)),
        ],
        out_shape=[
            jax.ShapeDtypeStruct((NN, FF), jnp.float32),
            jax.ShapeDtypeStruct((NN, FF), jnp.float32),
            jax.ShapeDtypeStruct((GG, FF), jnp.float32),
        ],
        scratch_shapes=[pltpu.VMEM((GG, FF), jnp.float32)],
    )(p1, p1, h1, dinv, b1, bat, rid, roots1, w2a, w2b)


def _tcc_body(a0_ref, a1_ref, h2l_ref, dinv_ref, b2_ref, bat_ref, r2_ref,
              out_ref, sum_ref, cnt_ref):
    i = pl.program_id(0)

    @pl.when(i == 0)
    def _():
        sum_ref[...] = jnp.zeros((GG, FF), jnp.float32)
        cnt_ref[...] = jnp.zeros((GG, 1), jnp.float32)

    dinv = dinv_ref[...]
    accs = (a0_ref[...] + a1_ref[...]).reshape(RB, FF)
    c2 = dinv * accs + dinv * dinv * h2l_ref[...] + b2_ref[...]
    relu2 = jnp.maximum(c2, 0.0)
    bat = bat_ref[...]
    bmat = (bat == lax.broadcasted_iota(jnp.int32, (RB, GG), 1)).astype(jnp.float32)
    sum_ref[...] += lax.dot_general(bmat, relu2, (((0,), (0,)), ((), ())),
                                    preferred_element_type=jnp.float32)
    ones_col = jnp.ones((RB, 1), jnp.float32)
    cnt_ref[...] += lax.dot_general(bmat, ones_col, (((0,), (0,)), ((), ())),
                                    preferred_element_type=jnp.float32)

    @pl.when(i == NBLK - 1)
    def _():
        cnt = cnt_ref[...]
        mean = sum_ref[...] / jnp.maximum(cnt, 1.0)
        rootp = jnp.where(cnt > 0.0, r2_ref[...], 0.0)
        out_ref[...] = jnp.concatenate([mean, rootp], axis=1)


@jax.jit
def _tc_c(p2, h2l, dinv, b2, bat, roots2):
    return pl.pallas_call(
        _tcc_body,
        grid=(NBLK,),
        in_specs=[
            pl.BlockSpec((1, RB, FF), lambda i: (0, i, 0)),
            pl.BlockSpec((1, RB, FF), lambda i: (1, i, 0)),
            pl.BlockSpec((RB, FF), lambda i: (i, 0)),
            pl.BlockSpec((RB, 1), lambda i: (i, 0)),
            pl.BlockSpec((1, FF), lambda i: (0, 0)),
            pl.BlockSpec((RB, 1), lambda i: (i, 0)),
            pl.BlockSpec((GG, FF), lambda i: (0, 0)),
        ],
        out_specs=pl.BlockSpec((GG, 2 * FF), lambda i: (0, 0)),
        out_shape=jax.ShapeDtypeStruct((GG, 2 * FF), jnp.float32),
        scratch_shapes=[
            pltpu.VMEM((GG, FF), jnp.float32),
            pltpu.VMEM((GG, 1), jnp.float32),
        ],
    )(p2, p2, h2l, dinv, b2, bat, roots2)


def kernel(x, edge_index, root_index, batch, W1, b1, W2, b2):
    x = x.astype(jnp.float32)
    ei3 = edge_index.reshape(2, NTILES * CPT, KC)

    degp = _sc_deg(ei3).reshape(2, NP_, 1)
    rid = root_index.reshape(GG, 1)
    bat = batch.reshape(NN, 1)

    h1, g1, dinv, roots1 = _tc_a(x, W1, degp, rid)

    p1 = _sc_spmm(g1, ei3).reshape(2, NP_, FF)
    g2, h2l, roots2 = _tc_b(p1, h1, dinv, b1.reshape(1, FF), bat, rid, roots1,
                            W2[:FF], W2[FF:])

    p2 = _sc_spmm(g2, ei3).reshape(2, NP_, FF)
    out = _tc_c(p2, h2l, dinv, b2.reshape(1, FF), bat, roots2)
    return out


# speedup vs baseline: 2.9899x; 1.0093x over previous
import functools

import jax
import jax.numpy as jnp
from jax import lax
from jax.experimental import pallas as pl
from jax.experimental.pallas import tpu as pltpu
from jax.experimental.pallas import tpu_sc as plsc

NN = 10000
EE = 320000
FF = 128
GG = 64
NP_ = 10240
KC = 80
NTILES = 32
CPT = 125
RB = 2000
NBLK = NN // RB


def _mesh():
    return plsc.VectorSubcoreMesh(core_axis_name="c", subcore_axis_name="s")


def _deg_body(dst_hbm, out_hbm, didx, ones_v, stage_v, acc, sa, sb):
    c = lax.axis_index("c")
    s = lax.axis_index("s")
    wid = s * 2 + c

    def fill_z(i, _):
        stage_v[pl.ds(i * 16, 16)] = jnp.zeros((16,), jnp.float32)
        return 0
    lax.fori_loop(0, 40, fill_z, 0)

    def fill_o(i, _):
        ones_v[pl.ds(i * 16, 16)] = jnp.full((16,), 1.0, jnp.float32)
        return 0
    lax.fori_loop(0, KC // 16, fill_o, 0)

    pltpu.sync_copy(stage_v, acc.at[pl.ds(s * 640, 640)])
    base = wid * CPT
    ab = pl.multiple_of(
        jnp.minimum((base // 8) * 8, NTILES * CPT - 136), 8)
    off = base - ab
    pltpu.sync_copy(dst_hbm.at[1, pl.ds(ab, 136)], didx)
    plsc.subcore_barrier()

    def step(p, _):
        da = pltpu.async_copy(ones_v, acc.at[didx.at[off + 2 * p]], sa, add=True)
        db = pltpu.async_copy(ones_v, acc.at[didx.at[off + 2 * p + 1]], sb, add=True)
        da.wait()
        db.wait()
        return 0
    lax.fori_loop(0, CPT // 2, step, 0)
    dl = pltpu.async_copy(ones_v, acc.at[didx.at[off + CPT - 1]], sa, add=True)
    dl.wait()

    plsc.subcore_barrier()
    pltpu.sync_copy(acc.at[pl.ds(s * 640, 640)], stage_v)
    pltpu.sync_copy(stage_v, out_hbm.at[pl.ds(c * NP_ + s * 640, 640)])


@jax.jit
def _sc_deg(dst2d):
    k = pl.kernel(
        _deg_body,
        out_type=jax.ShapeDtypeStruct((2 * NP_,), jnp.float32),
        mesh=_mesh(),
        scratch_types=[
            pltpu.VMEM((136, KC), jnp.int32),
            pltpu.VMEM((KC,), jnp.float32),
            pltpu.VMEM((640,), jnp.float32),
            pltpu.VMEM_SHARED((NP_,), jnp.float32),
            pltpu.SemaphoreType.DMA,
            pltpu.SemaphoreType.DMA,
        ],
    )
    return k(dst2d)


def _spmm_body(g_hbm, src_hbm, out_hbm, sidx, didx, rowsa, rowsb,
               acc, sga, sgb, ssa, ssb):
    dst_hbm = src_hbm
    c = lax.axis_index("c")
    s = lax.axis_index("s")
    wid = s * 2 + c

    def fill_z(i, _):
        r = i // 8
        l = i - r * 8
        rowsa[r, pl.ds(l * 16, 16)] = jnp.zeros((16,), jnp.float32)
        return 0
    lax.fori_loop(0, KC * 8, fill_z, 0)

    def zstripe(t, _):
        pltpu.sync_copy(rowsa, acc.at[pl.ds(s * 640 + t * KC, KC)])
        return 0
    lax.fori_loop(0, 640 // KC, zstripe, 0)
    plsc.subcore_barrier()

    for h in (0, 1):
        base = wid * CPT + h * 62
        ab = pl.multiple_of(
            jnp.minimum((base // 8) * 8, NTILES * CPT - 72), 8)
        off = base - ab
        pltpu.sync_copy(src_hbm.at[0, pl.ds(ab, 72)], sidx)
        pltpu.sync_copy(src_hbm.at[1, pl.ds(ab, 72)], didx)
        pltpu.async_copy(g_hbm.at[sidx.at[off]], rowsa, sga)
        pltpu.async_copy(g_hbm.at[sidx.at[off + 1]], rowsb, sgb)

        def step(p, _):
            t0 = off + 2 * p
            pltpu.make_async_copy(g_hbm.at[sidx.at[t0]], rowsa, sga).wait()
            sa = pltpu.async_copy(rowsa, acc.at[didx.at[t0]], ssa, add=True)
            pltpu.make_async_copy(g_hbm.at[sidx.at[t0 + 1]], rowsb, sgb).wait()
            sb = pltpu.async_copy(rowsb, acc.at[didx.at[t0 + 1]], ssb, add=True)
            sa.wait()

            @pl.when(p < 30 + h)
            def _():
                pltpu.async_copy(g_hbm.at[sidx.at[t0 + 2]], rowsa, sga)

            sb.wait()

            @pl.when(p < 30)
            def _():
                pltpu.async_copy(g_hbm.at[sidx.at[t0 + 3]], rowsb, sgb)

            return 0
        lax.fori_loop(0, 31, step, 0)
        if h == 1:
            pltpu.make_async_copy(g_hbm.at[sidx.at[off + 62]], rowsa, sga).wait()
            sl = pltpu.async_copy(rowsa, acc.at[didx.at[off + 62]], ssa, add=True)
            sl.wait()

    plsc.subcore_barrier()

    def wout(q, _):
        r0 = s * 640 + q * (2 * KC)
        o0 = c * NP_ + r0
        ia = pltpu.async_copy(acc.at[pl.ds(r0, KC)], rowsa, sga)
        ib = pltpu.async_copy(acc.at[pl.ds(r0 + KC, KC)], rowsb, sgb)
        ia.wait()
        oa = pltpu.async_copy(rowsa, out_hbm.at[pl.ds(o0, KC)], ssa)
        ib.wait()
        ob = pltpu.async_copy(rowsb, out_hbm.at[pl.ds(o0 + KC, KC)], ssb)
        oa.wait()
        ob.wait()
        return 0
    lax.fori_loop(0, 320 // KC, wout, 0)


@jax.jit
def _sc_spmm(g, ei3):
    k = pl.kernel(
        _spmm_body,
        out_type=jax.ShapeDtypeStruct((2 * NP_, FF), jnp.float32),
        mesh=_mesh(),
        scratch_types=[
            pltpu.VMEM((72, KC), jnp.int32),
            pltpu.VMEM((72, KC), jnp.int32),
            pltpu.VMEM((KC, FF), jnp.float32),
            pltpu.VMEM((KC, FF), jnp.float32),
            pltpu.VMEM_SHARED((NP_, FF), jnp.float32),
            pltpu.SemaphoreType.DMA,
            pltpu.SemaphoreType.DMA,
            pltpu.SemaphoreType.DMA,
            pltpu.SemaphoreType.DMA,
        ],
    )
    return k(g, ei3)


def _tca_body(x_ref, w1_ref, d0_ref, d1_ref, rid_ref,
              h1_ref, g1_ref, dinv_ref, r1_ref):
    i = pl.program_id(0)
    xb = x_ref[...]
    h1 = jnp.dot(xb, w1_ref[...], preferred_element_type=jnp.float32)
    deg = (d0_ref[...] + d1_ref[...]).reshape(RB, 1) + 1.0
    dinv = lax.rsqrt(deg)
    h1_ref[...] = h1
    dinv_ref[...] = dinv
    g1_ref[...] = h1 * dinv
    rid = rid_ref[...]
    col = lax.broadcasted_iota(jnp.int32, (GG, RB), 1) + i * RB
    pmat = (rid == col).astype(jnp.float32)

    @pl.when(i == 0)
    def _():
        r1_ref[...] = jnp.zeros((GG, FF), jnp.float32)

    r1_ref[...] += jnp.dot(pmat, xb, preferred_element_type=jnp.float32)


@jax.jit
def _tc_a(x, W1, degp, rid):
    return pl.pallas_call(
        _tca_body,
        grid=(NBLK,),
        in_specs=[
            pl.BlockSpec((RB, FF), lambda i: (i, 0)),
            pl.BlockSpec((FF, FF), lambda i: (0, 0)),
            pl.BlockSpec((1, RB, 1), lambda i: (0, i, 0)),
            pl.BlockSpec((1, RB, 1), lambda i: (1, i, 0)),
            pl.BlockSpec((GG, 1), lambda i: (0, 0)),
        ],
        out_specs=[
            pl.BlockSpec((RB, FF), lambda i: (i, 0)),
            pl.BlockSpec((RB, FF), lambda i: (i, 0)),
            pl.BlockSpec((RB, 1), lambda i: (i, 0)),
            pl.BlockSpec((GG, FF), lambda i: (0, 0)),
        ],
        out_shape=[
            jax.ShapeDtypeStruct((NN, FF), jnp.float32),
            jax.ShapeDtypeStruct((NN, FF), jnp.float32),
            jax.ShapeDtypeStruct((NN, 1), jnp.float32),
            jax.ShapeDtypeStruct((GG, FF), jnp.float32),
        ],
    )(x, W1, degp, degp, rid)


def _tcb_body(a0_ref, a1_ref, h1_ref, dinv_ref, b1_ref, bat_ref, rid_ref,
              r1_ref, w2a_ref, w2b_ref,
              g2_ref, h2l_ref, r2_ref, r1w_ref):
    i = pl.program_id(0)

    @pl.when(i == 0)
    def _():
        r1w_ref[...] = jnp.dot(jnp.maximum(r1_ref[...], 0.0), w2b_ref[...],
                               preferred_element_type=jnp.float32)
        r2_ref[...] = jnp.zeros((GG, FF), jnp.float32)

    dinv = dinv_ref[...]
    accs = (a0_ref[...] + a1_ref[...]).reshape(RB, FF)
    c1 = dinv * accs + dinv * dinv * h1_ref[...] + b1_ref[...]
    relu1 = jnp.maximum(c1, 0.0)
    bat = bat_ref[...]
    bmat = (bat == lax.broadcasted_iota(jnp.int32, (RB, GG), 1)).astype(jnp.float32)
    h2 = (jnp.dot(relu1, w2a_ref[...], preferred_element_type=jnp.float32)
          + jnp.dot(bmat, r1w_ref[...], preferred_element_type=jnp.float32))
    h2l_ref[...] = h2
    g2_ref[...] = h2 * dinv

    rid = rid_ref[...]
    col = lax.broadcasted_iota(jnp.int32, (GG, RB), 1) + i * RB
    pmat = (rid == col).astype(jnp.float32)
    r2_ref[...] += jnp.dot(pmat, c1, preferred_element_type=jnp.float32)


@jax.jit
def _tc_b(p1, h1, dinv, b1, bat, rid, roots1, w2a, w2b):
    return pl.pallas_call(
        _tcb_body,
        grid=(NBLK,),
        in_specs=[
            pl.BlockSpec((1, RB, FF), lambda i: (0, i, 0)),
            pl.BlockSpec((1, RB, FF), lambda i: (1, i, 0)),
            pl.BlockSpec((RB, FF), lambda i: (i, 0)),
            pl.BlockSpec((RB, 1), lambda i: (i, 0)),
            pl.BlockSpec((1, FF), lambda i: (0, 0)),
            pl.BlockSpec((RB, 1), lambda i: (i, 0)),
            pl.BlockSpec((GG, 1), lambda i: (0, 0)),
            pl.BlockSpec((GG, FF), lambda i: (0, 0)),
            pl.BlockSpec((FF, FF), lambda i: (0, 0)),
            pl.BlockSpec((FF, FF), lambda i: (0, 0)),
        ],
        out_specs=[
            pl.BlockSpec((RB, FF), lambda i: (i, 0)),
            pl.BlockSpec((RB, FF), lambda i: (i, 0)),
            pl.BlockSpec((GG, FF), lambda i: (0, 0)),
        ],
        out_shape=[
            jax.ShapeDtypeStruct((NN, FF), jnp.float32),
            jax.ShapeDtypeStruct((NN, FF), jnp.float32),
            jax.ShapeDtypeStruct((GG, FF), jnp.float32),
        ],
        scratch_shapes=[pltpu.VMEM((GG, FF), jnp.float32)],
    )(p1, p1, h1, dinv, b1, bat, rid, roots1, w2a, w2b)


def _tcc_body(a0_ref, a1_ref, h2l_ref, dinv_ref, b2_ref, bat_ref, r2_ref,
              out_ref, sum_ref, cnt_ref):
    i = pl.program_id(0)

    @pl.when(i == 0)
    def _():
        sum_ref[...] = jnp.zeros((GG, FF), jnp.float32)
        cnt_ref[...] = jnp.zeros((GG, 1), jnp.float32)

    dinv = dinv_ref[...]
    accs = (a0_ref[...] + a1_ref[...]).reshape(RB, FF)
    c2 = dinv * accs + dinv * dinv * h2l_ref[...] + b2_ref[...]
    relu2 = jnp.maximum(c2, 0.0)
    bat = bat_ref[...]
    bmat = (bat == lax.broadcasted_iota(jnp.int32, (RB, GG), 1)).astype(jnp.float32)
    sum_ref[...] += lax.dot_general(bmat, relu2, (((0,), (0,)), ((), ())),
                                    preferred_element_type=jnp.float32)
    ones_col = jnp.ones((RB, 1), jnp.float32)
    cnt_ref[...] += lax.dot_general(bmat, ones_col, (((0,), (0,)), ((), ())),
                                    preferred_element_type=jnp.float32)

    @pl.when(i == NBLK - 1)
    def _():
        cnt = cnt_ref[...]
        mean = sum_ref[...] / jnp.maximum(cnt, 1.0)
        rootp = jnp.where(cnt > 0.0, r2_ref[...], 0.0)
        out_ref[...] = jnp.concatenate([mean, rootp], axis=1)


@jax.jit
def _tc_c(p2, h2l, dinv, b2, bat, roots2):
    return pl.pallas_call(
        _tcc_body,
        grid=(NBLK,),
        in_specs=[
            pl.BlockSpec((1, RB, FF), lambda i: (0, i, 0)),
            pl.BlockSpec((1, RB, FF), lambda i: (1, i, 0)),
            pl.BlockSpec((RB, FF), lambda i: (i, 0)),
            pl.BlockSpec((RB, 1), lambda i: (i, 0)),
            pl.BlockSpec((1, FF), lambda i: (0, 0)),
            pl.BlockSpec((RB, 1), lambda i: (i, 0)),
            pl.BlockSpec((GG, FF), lambda i: (0, 0)),
        ],
        out_specs=pl.BlockSpec((GG, 2 * FF), lambda i: (0, 0)),
        out_shape=jax.ShapeDtypeStruct((GG, 2 * FF), jnp.float32),
        scratch_shapes=[
            pltpu.VMEM((GG, FF), jnp.float32),
            pltpu.VMEM((GG, 1), jnp.float32),
        ],
    )(p2, p2, h2l, dinv, b2, bat, roots2)


def kernel(x, edge_index, root_index, batch, W1, b1, W2, b2):
    x = x.astype(jnp.float32)
    ei3 = edge_index.reshape(2, NTILES * CPT, KC)

    degp = _sc_deg(ei3).reshape(2, NP_, 1)
    rid = root_index.reshape(GG, 1)
    bat = batch.reshape(NN, 1)

    h1, g1, dinv, roots1 = _tc_a(x, W1, degp, rid)

    p1 = _sc_spmm(g1, ei3).reshape(2, NP_, FF)
    g2, h2l, roots2 = _tc_b(p1, h1, dinv, b1.reshape(1, FF), bat, rid, roots1,
                            W2[:FF], W2[FF:])

    p2 = _sc_spmm(g2, ei3).reshape(2, NP_, FF)
    out = _tc_c(p2, h2l, dinv, b2.reshape(1, FF), bat, roots2)
    return out
